# Initial kernel scaffold; baseline (speedup 1.0000x reference)
import jax
import jax.numpy as jnp
from jax.experimental import pallas as pl


def kernel(x, Wq, bq, Wk, bk, Wv, bv, Wo, bo):
    raise NotImplementedError("write your pallas kernel here")


# speedup vs baseline: 8.4970x; 8.4970x over previous
import functools
import math

import jax
import jax.numpy as jnp
from jax.experimental import pallas as pl

EMBED = 1024
K = 32
_SCALE = 1.0 / math.sqrt(float(EMBED))


def _qkv_kernel(x_ref, wt_ref, b_ref, q_ref, k_ref, v_ref):
    x = x_ref[0].astype(jnp.bfloat16)
    wt = wt_ref[...].astype(jnp.bfloat16)
    qkv = jax.lax.dot_general(x, wt, (((1,), (0,)), ((), ())),
                              preferred_element_type=jnp.float32)
    qkv = qkv + b_ref[...]
    d = x.shape[1]
    q_ref[0] = qkv[:, 0:d]
    k_ref[0] = qkv[:, d:2 * d]
    v_ref[0] = qkv[:, 2 * d:3 * d]


def _attn_kernel(q_ref, k_ref, v_ref, wot_ref, bo_ref, o_ref):
    q = q_ref[0].astype(jnp.bfloat16)
    kk = k_ref[0].astype(jnp.bfloat16)
    s = jax.lax.dot_general(q, kk, (((1,), (1,)), ((), ())),
                            preferred_element_type=jnp.float32)
    s = s * _SCALE

    bits = jax.lax.bitcast_convert_type(s, jnp.int32)
    key = jnp.where(bits >= 0, bits, bits ^ jnp.int32(0x7FFFFFFF))

    nneg = jnp.sum((key >= 0).astype(jnp.int32), axis=1, keepdims=True)
    t = jnp.where(nneg >= K, jnp.int32(0), jnp.int32(-2147483648))
    for bit in range(30, -1, -1):
        cand = t | jnp.int32(1 << bit)
        cnt = jnp.sum((key >= cand).astype(jnp.int32), axis=1, keepdims=True)
        t = jnp.where(cnt >= K, cand, t)

    mask = key >= t
    m = jnp.max(s, axis=1, keepdims=True)
    p = jnp.where(mask, jnp.exp(s - m), 0.0)
    z = jnp.sum(p, axis=1, keepdims=True)
    w = (p / z).astype(jnp.bfloat16)

    out = jax.lax.dot_general(w, v_ref[0].astype(jnp.bfloat16),
                              (((1,), (0,)), ((), ())),
                              preferred_element_type=jnp.float32)
    res = jax.lax.dot_general(out.astype(jnp.bfloat16),
                              wot_ref[...].astype(jnp.bfloat16),
                              (((1,), (0,)), ((), ())),
                              preferred_element_type=jnp.float32)
    o_ref[0] = res + bo_ref[...]


@functools.partial(jax.jit, static_argnames=("blk_qkv", "blk"))
def _run(x, Wq, bq, Wk, bk, Wv, bv, Wo, bo, blk_qkv=512, blk=256):
    B, S, D = x.shape
    wt = jnp.concatenate([Wq, Wk, Wv], axis=0).T
    bqkv = jnp.concatenate([bq, bk, bv])[None, :]

    shape_sd = jax.ShapeDtypeStruct((B, S, D), jnp.float32)
    q, k, v = pl.pallas_call(
        _qkv_kernel,
        grid=(B, S // blk_qkv),
        in_specs=[
            pl.BlockSpec((1, blk_qkv, D), lambda b, i: (b, i, 0)),
            pl.BlockSpec((D, 3 * D), lambda b, i: (0, 0)),
            pl.BlockSpec((1, 3 * D), lambda b, i: (0, 0)),
        ],
        out_specs=(
            pl.BlockSpec((1, blk_qkv, D), lambda b, i: (b, i, 0)),
            pl.BlockSpec((1, blk_qkv, D), lambda b, i: (b, i, 0)),
            pl.BlockSpec((1, blk_qkv, D), lambda b, i: (b, i, 0)),
        ),
        out_shape=(shape_sd, shape_sd, shape_sd),
    )(x, wt, bqkv)

    wot = Wo.T
    result = pl.pallas_call(
        _attn_kernel,
        grid=(B, S // blk),
        in_specs=[
            pl.BlockSpec((1, blk, D), lambda b, i: (b, i, 0)),
            pl.BlockSpec((1, S, D), lambda b, i: (b, 0, 0)),
            pl.BlockSpec((1, S, D), lambda b, i: (b, 0, 0)),
            pl.BlockSpec((D, D), lambda b, i: (0, 0)),
            pl.BlockSpec((1, D), lambda b, i: (0, 0)),
        ],
        out_specs=pl.BlockSpec((1, blk, D), lambda b, i: (b, i, 0)),
        out_shape=jax.ShapeDtypeStruct((B, S, D), jnp.float32),
    )(q, k, v, wot, bo[None, :])
    return result


def kernel(x, Wq, bq, Wk, bk, Wv, bv, Wo, bo):
    return _run(x, Wq, bq, Wk, bk, Wv, bv, Wo, bo)
